# baseline (speedup 1.0000x reference)
import functools

import jax
import jax.numpy as jnp
from jax import lax
from jax.experimental import pallas as pl
from jax.experimental.pallas import tpu as pltpu
from jax.experimental.pallas import tpu_sc as plsc

B, A, S, C = 64, 3, 52, 80
NC = 5 + C
N = B * A * S * S
NF = N * NC
NW = 32
CW = N // NW
PLANE = S * S
CHUNK = 128
NCHUNK = (CW + CHUNK - 1) // CHUNK
NBUF = 4


def _log1p_of_exp_neg(a):
    u = jnp.exp(-a)
    z = u / (2.0 + u)
    z2 = z * z
    return z * (2.0 + z2 * (0.6666667 + z2 * (0.4 + z2 * (0.28571429
                + z2 * 0.22222222))))


def _sc_body(pred_hbm, tgt_hbm, anchw_hbm, anchh_hbm, out_hbm,
             idx_ref, dbuf, acc_ref, anchw_v, anchh_v,
             s0, s1, s2, s3, sa):
    sems = (s0, s1, s2, s3)
    cid = lax.axis_index("c")
    sid = lax.axis_index("s")
    wid = sid * 2 + cid
    base = wid * CW
    lane = lax.broadcasted_iota(jnp.int32, (16,), 0)

    pltpu.make_async_copy(anchw_hbm, anchw_v, sa).start()
    pltpu.make_async_copy(anchh_hbm, anchh_v, sa).start()
    pltpu.make_async_copy(anchw_hbm, anchw_v, sa).wait()
    pltpu.make_async_copy(anchh_hbm, anchh_v, sa).wait()

    for r in range(4):
        acc_ref[r] = jnp.zeros((16,), jnp.float32)

    def build(j, _):
        for k in range(8):
            cell = base + j * CHUNK + k * 16 + lane
            cell = jnp.minimum(cell, N - 1)
            f = cell * NC
            for q in range(5):
                idx_ref[q, j, 0, pl.ds(k * 16, 16)] = f + q
        return 0
    lax.fori_loop(0, NCHUNK, build, 0)

    def fire(j, b):
        pltpu.make_async_copy(pred_hbm.at[idx_ref.at[0, j, 0]],
                              dbuf.at[b, 0, 0], sems[b]).start()
        for q in range(5):
            pltpu.make_async_copy(tgt_hbm.at[idx_ref.at[q, j, 0]],
                                  dbuf.at[b, 1 + q, 0], sems[b]).start()

    def wait(j, b):
        pltpu.make_async_copy(pred_hbm.at[idx_ref.at[0, j, 0]],
                              dbuf.at[b, 0, 0], sems[b]).wait()
        for q in range(5):
            pltpu.make_async_copy(tgt_hbm.at[idx_ref.at[q, j, 0]],
                                  dbuf.at[b, 1 + q, 0], sems[b]).wait()

    def compute(j, b):
        for k in range(8):
            cell = base + j * CHUNK + k * 16 + lane
            valid = cell < N
            p0 = dbuf[b, 0, 0, pl.ds(k * 16, 16)]
            t0 = dbuf[b, 1, 0, pl.ds(k * 16, 16)]
            noobj_m = jnp.where((t0 == 0.0) & valid, 1.0, 0.0)
            noobj_terms = (jnp.maximum(p0, 0.0)
                           + _log1p_of_exp_neg(jnp.abs(p0)))
            t1 = dbuf[b, 2, 0, pl.ds(k * 16, 16)]
            t2 = dbuf[b, 3, 0, pl.ds(k * 16, 16)]
            t3 = dbuf[b, 4, 0, pl.ds(k * 16, 16)]
            t4 = dbuf[b, 5, 0, pl.ds(k * 16, 16)]
            obj = (t0 == 1.0) & valid
            obj_m = jnp.where(obj, 1.0, 0.0)
            s = base + j * CHUNK + k * 16
            sx = s % S
            sy = (s // S) % S
            a_sc = (s // PLANE) % A
            aw = anchw_v[a_sc]
            ah = anchh_v[a_sc]
            xs = sx + lane
            wrap = xs >= S
            x_i = jnp.where(wrap, xs - S, xs)
            y_i = jnp.where(wrap, sy + 1, sy)
            y_i = jnp.where(y_i >= S, y_i - S, y_i)
            xf = x_i.astype(jnp.float32)
            yf = y_i.astype(jnp.float32)
            bx = 1.0 / (1.0 + jnp.exp(-t1)) + xf
            by = 1.0 / (1.0 + jnp.exp(-t2)) + yf
            bw = jnp.exp(t3) * aw
            bh = jnp.exp(t4) * ah
            b1x1 = bx - bw * 0.5
            b1y1 = by - bh * 0.5
            b1x2 = bx + bw * 0.5
            b1y2 = by + bh * 0.5
            b2x1 = t1 - t3 * 0.5
            b2y1 = t2 - t4 * 0.5
            b2x2 = t1 + t3 * 0.5
            b2y2 = t2 + t4 * 0.5
            ix1 = jnp.maximum(b1x1, b2x1)
            iy1 = jnp.maximum(b1y1, b2y1)
            ix2 = jnp.minimum(b1x2, b2x2)
            iy2 = jnp.minimum(b1y2, b2y2)
            inter = (jnp.maximum(ix2 - ix1, 0.0)
                     * jnp.maximum(iy2 - iy1, 0.0))
            area1 = (b1x2 - b1x1) * (b1y2 - b1y1)
            area2 = (b2x2 - b2x1) * (b2y2 - b2y1)
            union = area1 + area2 - inter + 1e-6
            iou = inter / union
            obj_terms = (jnp.maximum(iou, 0.0) - iou * p0
                         + _log1p_of_exp_neg(jnp.abs(iou)))
            acc_ref[0] = acc_ref[0] + noobj_terms * noobj_m
            acc_ref[1] = acc_ref[1] + obj_terms * obj_m
            acc_ref[2] = acc_ref[2] + obj_m
            acc_ref[3] = acc_ref[3] + noobj_m

    for b in range(NBUF):
        fire(b, b)

    def group(g, _):
        for b in range(NBUF):
            j = g * NBUF + b

            @pl.when(j < NCHUNK)
            def _():
                wait(j, b)
                compute(j, b)

            @pl.when(j + NBUF < NCHUNK)
            def _():
                fire(j + NBUF, b)
        return 0

    lax.fori_loop(0, (NCHUNK + NBUF - 1) // NBUF, group, 0)

    pltpu.sync_copy(acc_ref, out_hbm.at[wid])


_sc_loss = functools.partial(
    pl.kernel,
    out_type=jax.ShapeDtypeStruct((NW, 4, 16), jnp.float32),
    mesh=plsc.VectorSubcoreMesh(core_axis_name="c", subcore_axis_name="s"),
    scratch_types=[
        pltpu.VMEM((5, NCHUNK, 1, CHUNK), jnp.int32),
        pltpu.VMEM((NBUF, 6, 1, CHUNK), jnp.float32),
        pltpu.VMEM((4, 16), jnp.float32),
        pltpu.VMEM((A, 16), jnp.float32),
        pltpu.VMEM((A, 16), jnp.float32),
    ] + [pltpu.SemaphoreType.DMA] * 5,
)(_sc_body)


@jax.jit
def kernel(predictions, targets, anchors):
    pred_f = predictions.reshape(NF)
    tgt_f = targets.reshape(NF)
    anch_w = jnp.broadcast_to(anchors[:, 0:1], (A, 16))
    anch_h = jnp.broadcast_to(anchors[:, 1:2], (A, 16))
    partials = _sc_loss(pred_f, tgt_f, anch_w, anch_h)
    sums = jnp.sum(partials, axis=(0, 2))
    no_obj_loss = sums[0] / sums[3]
    obj_loss = sums[1] / sums[2]
    return 0.5 * no_obj_loss + obj_loss


# speedup vs baseline: 2.0868x; 2.0868x over previous
import jax
import jax.numpy as jnp
from jax.experimental import pallas as pl

B, A, S, C = 64, 3, 52, 80
NC = 5 + C
R = A * S


def _loss_kernel(pred_ref, tgt_ref, an_ref, out_ref):
    ident = (jax.lax.broadcasted_iota(jnp.int32, (S, S), 0)
             == jax.lax.broadcasted_iota(jnp.int32, (S, S), 1)
             ).astype(jnp.float32)
    dn = (((1,), (0,)), ((), ()))
    pb = pred_ref[...].reshape(R, S, NC)[:, :, :8]
    tb = tgt_ref[...].reshape(R, S, NC)[:, :, :8]
    pt = jax.lax.dot_general(pb, ident, dn, preferred_element_type=jnp.float32)
    tt = jax.lax.dot_general(tb, ident, dn, preferred_element_type=jnp.float32)
    p0 = pt[:, 0, :]
    t0 = tt[:, 0, :]
    t1 = tt[:, 1, :]
    t2 = tt[:, 2, :]
    t3 = tt[:, 3, :]
    t4 = tt[:, 4, :]

    row = jax.lax.broadcasted_iota(jnp.int32, (R, S), 0)
    a_idx = row // S
    y = (row % S).astype(jnp.float32)
    x = jax.lax.broadcasted_iota(jnp.int32, (R, S), 1).astype(jnp.float32)

    obj_m = (t0 == 1.0).astype(jnp.float32)
    noobj_m = (t0 == 0.0).astype(jnp.float32)

    noobj_terms = (jnp.maximum(p0, 0.0) - p0 * t0
                   + jnp.log1p(jnp.exp(-jnp.abs(p0))))

    aw = jnp.where(a_idx == 0, an_ref[0, 0],
                   jnp.where(a_idx == 1, an_ref[1, 0], an_ref[2, 0]))
    ah = jnp.where(a_idx == 0, an_ref[0, 1],
                   jnp.where(a_idx == 1, an_ref[1, 1], an_ref[2, 1]))
    bx = jax.nn.sigmoid(t1) + x
    by = jax.nn.sigmoid(t2) + y
    bw = jnp.exp(t3) * aw
    bh = jnp.exp(t4) * ah

    b1x1 = bx - bw * 0.5
    b1y1 = by - bh * 0.5
    b1x2 = bx + bw * 0.5
    b1y2 = by + bh * 0.5
    b2x1 = t1 - t3 * 0.5
    b2y1 = t2 - t4 * 0.5
    b2x2 = t1 + t3 * 0.5
    b2y2 = t2 + t4 * 0.5
    ix1 = jnp.maximum(b1x1, b2x1)
    iy1 = jnp.maximum(b1y1, b2y1)
    ix2 = jnp.minimum(b1x2, b2x2)
    iy2 = jnp.minimum(b1y2, b2y2)
    inter = (jnp.clip(ix2 - ix1, 0.0, None) * jnp.clip(iy2 - iy1, 0.0, None))
    area1 = (b1x2 - b1x1) * (b1y2 - b1y1)
    area2 = (b2x2 - b2x1) * (b2y2 - b2y1)
    union = area1 + area2 - inter + 1e-6
    iou = inter / union
    obj_terms = (jnp.maximum(iou, 0.0) - iou * p0
                 + jnp.log1p(jnp.exp(-jnp.abs(iou))))

    noobj_row = jnp.sum(noobj_terms * noobj_m, axis=0, keepdims=True)
    obj_row = jnp.sum(obj_terms * obj_m, axis=0, keepdims=True)
    k_row = jnp.sum(obj_m, axis=0, keepdims=True)
    n_row = jnp.sum(noobj_m, axis=0, keepdims=True)
    rows = jnp.concatenate(
        [noobj_row, obj_row, k_row, n_row,
         jnp.zeros((4, S), dtype=jnp.float32)], axis=0)
    out_ref[...] = jnp.zeros((8, 128), jnp.float32)
    out_ref[:, 0:S] = rows


@jax.jit
def kernel(predictions, targets, anchors):
    anch = jnp.zeros((8, 128), jnp.float32).at[:A, :2].set(anchors)

    partials = pl.pallas_call(
        _loss_kernel,
        grid=(B,),
        in_specs=[
            pl.BlockSpec((1, A, S, S, NC), lambda i: (i, 0, 0, 0, 0)),
            pl.BlockSpec((1, A, S, S, NC), lambda i: (i, 0, 0, 0, 0)),
            pl.BlockSpec((8, 128), lambda i: (0, 0)),
        ],
        out_specs=pl.BlockSpec((None, 8, 128), lambda i: (i, 0, 0)),
        out_shape=jax.ShapeDtypeStruct((B, 8, 128), jnp.float32),
    )(predictions, targets, anch)

    sums = jnp.sum(partials, axis=(0, 2))
    no_obj_loss = sums[0] / sums[3]
    obj_loss = sums[1] / sums[2]
    return 0.5 * no_obj_loss + obj_loss
